# baseline (speedup 1.0000x reference)
import functools

import jax
import jax.numpy as jnp
from jax import lax
from jax.experimental import pallas as pl
from jax.experimental.pallas import tpu as pltpu
from jax.experimental.pallas import tpu_sc as plsc

B, N, H, W = 2, 2048, 128, 3
EPS = 1e-8
EPS2 = EPS * EPS
C = 64
NW = 32
LANES = 16
NCG = H // LANES
POFFS = (1, 2, 3)
OFFS = (-3, -2, -1, 1, 2, 3)
VROWS = C + 16
UROWS = C + 6
XROWS = C + 16


def _rsqrt_nr(s):
    i = lax.bitcast_convert_type(s, jnp.int32)
    y = lax.bitcast_convert_type(jnp.int32(0x5F3759DF) - (i >> 1), jnp.float32)
    for _ in range(2):
        y = y * (1.5 - 0.5 * s * y * y)
    return y


def _inv_norm(s):
    return _rsqrt_nr(jnp.maximum(s, EPS2))


def _sigmoid(z):
    return 1.0 / (1.0 + jnp.exp(-z))


def _sc_body(vec_hbm, x_hbm, w_hbm, q_hbm, k_hbm, vecl, xl, ul, ql, kl, wl):
    wid = lax.axis_index("s") * 2 + lax.axis_index("c")
    n0 = wid * C
    sv = jnp.clip(n0 - 8, 0, N - VROWS)
    sx = jnp.clip(n0 - 8, 0, N - XROWS)

    pltpu.sync_copy(w_hbm, wl)

    def chunk_body(it, _):
        bb = it * N
        pltpu.sync_copy(
            vec_hbm.at[pl.ds(pl.multiple_of(3 * (bb + sv), 8), 3 * VROWS)],
            vecl)
        pltpu.sync_copy(
            x_hbm.at[pl.ds(pl.multiple_of(bb + sx, 8), XROWS)], xl)

        def phase_a(ii, _):
            n = n0 - 3 + ii
            r = jnp.clip(n - sv, 0, VROWS - 1)
            vi_ok = jnp.where((n >= 0) & (n < N), 1.0, 0.0)

            def ch_a(c, _):
                cs = c * LANES
                sl = pl.ds(cs, LANES)
                vix = vecl[3 * r, sl]
                viy = vecl[3 * r + 1, sl]
                viz = vecl[3 * r + 2, sl]
                ux = jnp.zeros((LANES,), jnp.float32)
                uy = jnp.zeros((LANES,), jnp.float32)
                uz = jnp.zeros((LANES,), jnp.float32)
                for o in OFFS:
                    n2 = n + o
                    r2 = jnp.clip(n2 - sv, 0, VROWS - 1)
                    bx = vecl[3 * r2, sl] - vix
                    by = vecl[3 * r2 + 1, sl] - viy
                    bz = vecl[3 * r2 + 2, sl] - viz
                    s = bx * bx + by * by + bz * bz
                    ok = vi_ok * jnp.where((n2 >= 0) & (n2 < N), 1.0, 0.0)
                    f = _inv_norm(s) * ok
                    ux = ux + bx * f
                    uy = uy + by * f
                    uz = uz + bz * f
                ul[4 * ii, sl] = ux
                ul[4 * ii + 1, sl] = uy
                ul[4 * ii + 2, sl] = uz
                s_u = ux * ux + uy * uy + uz * uz
                ul[4 * ii + 3, sl] = _inv_norm(s_u)
                zero = jnp.zeros((LANES,), jnp.float32)
                ql[ii, sl] = zero
                kl[ii, sl] = zero
                return 0

            lax.fori_loop(0, NCG, ch_a, 0, unroll=False)
            return 0

        lax.fori_loop(0, UROWS, phase_a, 0, unroll=False)

        def phase_b(c, _):
            cs = c * LANES
            sl = pl.ds(cs, LANES)
            w0q = wl[0, sl]
            w1q = wl[1, sl]
            w2q = wl[2, sl]
            w0k = wl[3, sl]
            w1k = wl[4, sl]
            w2k = wl[5, sl]

            def node_b(i, _):
                n = n0 - 3 + i
                r = jnp.clip(n - sv, 0, VROWS - 1)
                vax = vecl[3 * r, sl]
                vay = vecl[3 * r + 1, sl]
                vaz = vecl[3 * r + 2, sl]
                uax = ul[4 * i, sl]
                uay = ul[4 * i + 1, sl]
                uaz = ul[4 * i + 2, sl]
                inv_ua = ul[4 * i + 3, sl]
                xa = xl[jnp.clip(n - sx, 0, XROWS - 1), sl]
                q_fwd = jnp.zeros((LANES,), jnp.float32)
                k_fwd = jnp.zeros((LANES,), jnp.float32)
                for o in POFFS:
                    nb = n + o
                    ok = jnp.where((n >= 0) & (nb < N), 1.0, 0.0)
                    rb = jnp.clip(nb - sv, 0, VROWS - 1)
                    bx = vecl[3 * rb, sl] - vax
                    by = vecl[3 * rb + 1, sl] - vay
                    bz = vecl[3 * rb + 2, sl] - vaz
                    s_e = bx * bx + by * by + bz * bz
                    inv_e = _inv_norm(s_e)
                    ex = bx * inv_e
                    ey = by * inv_e
                    ez = bz * inv_e
                    ib = i + o
                    ubx = ul[4 * ib, sl]
                    uby = ul[4 * ib + 1, sl]
                    ubz = ul[4 * ib + 2, sl]
                    inv_ub = ul[4 * ib + 3, sl]
                    d_a = uax * ex + uay * ey + uaz * ez
                    d_b = ubx * ex + uby * ey + ubz * ez
                    ang_ab = jnp.maximum(jnp.minimum(d_a * inv_ua, 1.0), -1.0)
                    ang_ba = jnp.maximum(
                        jnp.minimum(0.0 - d_b * inv_ub, 1.0), -1.0)
                    uapx = uax - d_a * ex
                    uapy = uay - d_a * ey
                    uapz = uaz - d_a * ez
                    ubpx = ubx - d_b * ex
                    ubpy = uby - d_b * ey
                    ubpz = ubz - d_b * ez
                    s_pa = uapx * uapx + uapy * uapy + uapz * uapz
                    s_pb = ubpx * ubpx + ubpy * ubpy + ubpz * ubpz
                    dotp = uapx * ubpx + uapy * ubpy + uapz * ubpz
                    spp = jnp.maximum(s_pa, EPS2) * jnp.maximum(s_pb, EPS2)
                    dih = dotp * _rsqrt_nr(spp)
                    dih = jnp.maximum(jnp.minimum(dih, 1.0), -1.0)
                    tq = dih * w1q + w2q
                    tk = dih * w1k + w2k
                    gq_ab = _sigmoid(tq + ang_ab * w0q)
                    gq_ba = _sigmoid(tq + ang_ba * w0q)
                    gk_ab = _sigmoid(tk + ang_ab * w0k)
                    gk_ba = _sigmoid(tk + ang_ba * w0k)
                    xb = xl[jnp.clip(nb - sx, 0, XROWS - 1), sl]
                    xa_ok = xa * ok
                    xb_ok = xb * ok
                    q_fwd = q_fwd + gq_ab * xb_ok
                    k_fwd = k_fwd + gk_ab * xb_ok
                    ql[ib, sl] = ql[ib, sl] + gq_ba * xa_ok
                    kl[ib, sl] = kl[ib, sl] + gk_ba * xa_ok
                ql[i, sl] = ql[i, sl] + q_fwd
                kl[i, sl] = kl[i, sl] + k_fwd
                return 0

            lax.fori_loop(0, UROWS - 3, node_b, 0, unroll=False)
            return 0

        lax.fori_loop(0, NCG, phase_b, 0, unroll=False)

        pltpu.sync_copy(ql.at[pl.ds(3, C)],
                        q_hbm.at[pl.ds(pl.multiple_of(bb + n0, 8), C)])
        pltpu.sync_copy(kl.at[pl.ds(3, C)],
                        k_hbm.at[pl.ds(pl.multiple_of(bb + n0, 8), C)])
        return 0

    lax.fori_loop(0, B, chunk_body, 0, unroll=False)


@jax.jit
def kernel(x_scalar, vec, w_angle_q, w_dih_q, b_q, w_angle_k, w_dih_k, b_k):
    vec_r = vec.reshape(B * N * 3, H)
    x_r = x_scalar.reshape(B * N, H)
    zrow = jnp.zeros_like(b_q)
    w_all = jnp.stack(
        [w_angle_q, w_dih_q, b_q, w_angle_k, w_dih_k, b_k, zrow, zrow])

    mesh = plsc.VectorSubcoreMesh(core_axis_name="c", subcore_axis_name="s")
    run = pl.kernel(
        _sc_body,
        out_type=(
            jax.ShapeDtypeStruct((B * N, H), jnp.float32),
            jax.ShapeDtypeStruct((B * N, H), jnp.float32),
        ),
        mesh=mesh,
        scratch_types=[
            pltpu.VMEM((3 * VROWS, H), jnp.float32),
            pltpu.VMEM((XROWS, H), jnp.float32),
            pltpu.VMEM((4 * UROWS, H), jnp.float32),
            pltpu.VMEM((UROWS, H), jnp.float32),
            pltpu.VMEM((UROWS, H), jnp.float32),
            pltpu.VMEM((8, H), jnp.float32),
        ],
    )
    q_r, k_r = run(vec_r, x_r, w_all)
    return q_r.reshape(B, N, H), k_r.reshape(B, N, H)


# speedup vs baseline: 1.4968x; 1.4968x over previous
import functools

import jax
import jax.numpy as jnp
from jax import lax
from jax.experimental import pallas as pl
from jax.experimental.pallas import tpu as pltpu
from jax.experimental.pallas import tpu_sc as plsc

B, N, H, W = 2, 2048, 128, 3
EPS = 1e-8
EPS2 = EPS * EPS
C = 64
NW = 32
LANES = 16
NCG = H // LANES
POFFS = (1, 2, 3)
OFFS = (-3, -2, -1, 1, 2, 3)
VROWS = C + 16
UROWS = C + 6
XROWS = C + 16


def _rsqrt_nr(s):
    i = lax.bitcast_convert_type(s, jnp.int32)
    y = lax.bitcast_convert_type(jnp.int32(0x5F3759DF) - (i >> 1), jnp.float32)
    for _ in range(2):
        y = y * (1.5 - 0.5 * s * y * y)
    return y


def _inv_norm(s):
    return _rsqrt_nr(jnp.maximum(s, EPS2))


def _sigmoid(z):
    return 1.0 / (1.0 + jnp.exp(-z))


def _sc_body(vec_hbm, x_hbm, w_hbm, q_hbm, k_hbm, vecl, xl, ul, ql, kl, wl):
    wid = lax.axis_index("s") * 2 + lax.axis_index("c")
    n0 = wid * C
    sv = jnp.clip(n0 - 8, 0, N - VROWS)
    sx = jnp.clip(n0 - 8, 0, N - XROWS)

    pltpu.sync_copy(w_hbm, wl)

    def chunk_body(it, _):
        bb = it * N
        pltpu.sync_copy(
            vec_hbm.at[pl.ds(pl.multiple_of(3 * (bb + sv), 8), 3 * VROWS)],
            vecl)
        pltpu.sync_copy(
            x_hbm.at[pl.ds(pl.multiple_of(bb + sx, 8), XROWS)], xl)

        def phase_a(ii, _):
            n = n0 - 3 + ii
            r = jnp.clip(n - sv, 0, VROWS - 1)
            vi_ok = jnp.where((n >= 0) & (n < N), 1.0, 0.0)

            def ch_a(c, _):
                cs = c * LANES
                sl = pl.ds(cs, LANES)
                vix = vecl[3 * r, sl]
                viy = vecl[3 * r + 1, sl]
                viz = vecl[3 * r + 2, sl]
                ux = jnp.zeros((LANES,), jnp.float32)
                uy = jnp.zeros((LANES,), jnp.float32)
                uz = jnp.zeros((LANES,), jnp.float32)
                for o in OFFS:
                    n2 = n + o
                    r2 = jnp.clip(n2 - sv, 0, VROWS - 1)
                    bx = vecl[3 * r2, sl] - vix
                    by = vecl[3 * r2 + 1, sl] - viy
                    bz = vecl[3 * r2 + 2, sl] - viz
                    s = bx * bx + by * by + bz * bz
                    ok = vi_ok * jnp.where((n2 >= 0) & (n2 < N), 1.0, 0.0)
                    f = _inv_norm(s) * ok
                    ux = ux + bx * f
                    uy = uy + by * f
                    uz = uz + bz * f
                ul[5 * ii, sl] = ux
                ul[5 * ii + 1, sl] = uy
                ul[5 * ii + 2, sl] = uz
                s_u = ux * ux + uy * uy + uz * uz
                ul[5 * ii + 3, sl] = _inv_norm(s_u)
                ul[5 * ii + 4, sl] = s_u
                return 0

            lax.fori_loop(0, NCG, ch_a, 0, unroll=False)
            return 0

        lax.fori_loop(0, UROWS, phase_a, 0, unroll=False)

        def phase_b(c, _):
            cs = c * LANES
            sl = pl.ds(cs, LANES)
            w0q = wl[0, sl]
            w1q = wl[1, sl]
            w2q = wl[2, sl]
            w0k = wl[3, sl]
            w1k = wl[4, sl]
            w2k = wl[5, sl]

            def node_b(i, _):
                n = n0 + i
                r = n - sv
                vix = vecl[3 * r, sl]
                viy = vecl[3 * r + 1, sl]
                viz = vecl[3 * r + 2, sl]
                iu = i + 3
                uix = ul[5 * iu, sl]
                uiy = ul[5 * iu + 1, sl]
                uiz = ul[5 * iu + 2, sl]
                inv_ui = ul[5 * iu + 3, sl]
                s_ui = ul[5 * iu + 4, sl]
                q_acc = jnp.zeros((LANES,), jnp.float32)
                k_acc = jnp.zeros((LANES,), jnp.float32)
                for o in OFFS:
                    n2 = n + o
                    ok = jnp.where((n2 >= 0) & (n2 < N), 1.0, 0.0)
                    r2 = jnp.clip(n2 - sv, 0, VROWS - 1)
                    bx = vecl[3 * r2, sl] - vix
                    by = vecl[3 * r2 + 1, sl] - viy
                    bz = vecl[3 * r2 + 2, sl] - viz
                    s_e = bx * bx + by * by + bz * bz
                    inv_e = _inv_norm(s_e)
                    iu2 = iu + o
                    ujx = ul[5 * iu2, sl]
                    ujy = ul[5 * iu2 + 1, sl]
                    ujz = ul[5 * iu2 + 2, sl]
                    s_uj = ul[5 * iu2 + 4, sl]
                    p_i = uix * bx + uiy * by + uiz * bz
                    p_j = ujx * bx + ujy * by + ujz * bz
                    d_i = p_i * inv_e
                    d_j = p_j * inv_e
                    ang = jnp.maximum(jnp.minimum(d_i * inv_ui, 1.0), -1.0)
                    s_pi = s_ui - d_i * d_i
                    s_pj = s_uj - d_j * d_j
                    uiuj = uix * ujx + uiy * ujy + uiz * ujz
                    dotp = uiuj - d_i * d_j
                    spp = jnp.maximum(s_pi, EPS2) * jnp.maximum(s_pj, EPS2)
                    dih = dotp * _rsqrt_nr(spp)
                    dih = jnp.maximum(jnp.minimum(dih, 1.0), -1.0)
                    zq = w2q + ang * w0q + dih * w1q
                    zk = w2k + ang * w0k + dih * w1k
                    gq = _sigmoid(zq)
                    gk = _sigmoid(zk)
                    xj = xl[jnp.clip(n2 - sx, 0, XROWS - 1), sl] * ok
                    q_acc = q_acc + gq * xj
                    k_acc = k_acc + gk * xj
                ql[i, sl] = q_acc
                kl[i, sl] = k_acc
                return 0

            lax.fori_loop(0, C, node_b, 0, unroll=False)
            return 0

        lax.fori_loop(0, NCG, phase_b, 0, unroll=False)

        pltpu.sync_copy(ql, q_hbm.at[pl.ds(pl.multiple_of(bb + n0, 8), C)])
        pltpu.sync_copy(kl, k_hbm.at[pl.ds(pl.multiple_of(bb + n0, 8), C)])
        return 0

    lax.fori_loop(0, B, chunk_body, 0, unroll=False)


@jax.jit
def kernel(x_scalar, vec, w_angle_q, w_dih_q, b_q, w_angle_k, w_dih_k, b_k):
    vec_r = vec.reshape(B * N * 3, H)
    x_r = x_scalar.reshape(B * N, H)
    zrow = jnp.zeros_like(b_q)
    w_all = jnp.stack(
        [w_angle_q, w_dih_q, b_q, w_angle_k, w_dih_k, b_k, zrow, zrow])

    mesh = plsc.VectorSubcoreMesh(core_axis_name="c", subcore_axis_name="s")
    run = pl.kernel(
        _sc_body,
        out_type=(
            jax.ShapeDtypeStruct((B * N, H), jnp.float32),
            jax.ShapeDtypeStruct((B * N, H), jnp.float32),
        ),
        mesh=mesh,
        scratch_types=[
            pltpu.VMEM((3 * VROWS, H), jnp.float32),
            pltpu.VMEM((XROWS, H), jnp.float32),
            pltpu.VMEM((5 * UROWS, H), jnp.float32),
            pltpu.VMEM((C, H), jnp.float32),
            pltpu.VMEM((C, H), jnp.float32),
            pltpu.VMEM((8, H), jnp.float32),
        ],
    )
    q_r, k_r = run(vec_r, x_r, w_all)
    return q_r.reshape(B, N, H), k_r.reshape(B, N, H)


# speedup vs baseline: 1.5322x; 1.0237x over previous
import functools

import jax
import jax.numpy as jnp
from jax import lax
from jax.experimental import pallas as pl
from jax.experimental.pallas import tpu as pltpu
from jax.experimental.pallas import tpu_sc as plsc

B, N, H, W = 2, 2048, 128, 3
EPS = 1e-8
EPS2 = EPS * EPS
C = 64
NW = 32
LANES = 16
NCG = H // LANES
POFFS = (1, 2, 3)
OFFS = (-3, -2, -1, 1, 2, 3)
VROWS = C + 16
UROWS = C + 6
XROWS = C + 16


def _rsqrt_nr(s):
    i = lax.bitcast_convert_type(s, jnp.int32)
    y = lax.bitcast_convert_type(jnp.int32(0x5F3759DF) - (i >> 1), jnp.float32)
    for _ in range(2):
        y = y * (1.5 - 0.5 * s * y * y)
    return y


def _inv_norm(s):
    return _rsqrt_nr(jnp.maximum(s, EPS2))


def _sigmoid(z):
    return 1.0 / (1.0 + jnp.exp(-z))


def _sc_body(vec_hbm, x_hbm, w_hbm, q_hbm, k_hbm, vecl, xl, ul, ql, kl, wl):
    wid = lax.axis_index("s") * 2 + lax.axis_index("c")
    n0 = wid * C
    sv = jnp.clip(n0 - 8, 0, N - VROWS)
    sx = jnp.clip(n0 - 8, 0, N - XROWS)

    pltpu.sync_copy(w_hbm, wl)

    def chunk_body(it, _):
        bb = it * N
        pltpu.sync_copy(
            vec_hbm.at[pl.ds(pl.multiple_of(3 * (bb + sv), 8), 3 * VROWS)],
            vecl)
        pltpu.sync_copy(
            x_hbm.at[pl.ds(pl.multiple_of(bb + sx, 8), XROWS)], xl)

        def phase_a(ii, _):
            n = n0 - 3 + ii
            r = jnp.clip(n - sv, 0, VROWS - 1)
            vi_ok = jnp.where((n >= 0) & (n < N), 1.0, 0.0)

            def ch_a(c, _):
                cs = c * LANES
                sl = pl.ds(cs, LANES)
                vix = vecl[3 * r, sl]
                viy = vecl[3 * r + 1, sl]
                viz = vecl[3 * r + 2, sl]
                ux = jnp.zeros((LANES,), jnp.float32)
                uy = jnp.zeros((LANES,), jnp.float32)
                uz = jnp.zeros((LANES,), jnp.float32)
                for o in OFFS:
                    n2 = n + o
                    r2 = jnp.clip(n2 - sv, 0, VROWS - 1)
                    bx = vecl[3 * r2, sl] - vix
                    by = vecl[3 * r2 + 1, sl] - viy
                    bz = vecl[3 * r2 + 2, sl] - viz
                    s = bx * bx + by * by + bz * bz
                    ok = vi_ok * jnp.where((n2 >= 0) & (n2 < N), 1.0, 0.0)
                    f = _inv_norm(s) * ok
                    ux = ux + bx * f
                    uy = uy + by * f
                    uz = uz + bz * f
                ul[5 * ii, sl] = ux
                ul[5 * ii + 1, sl] = uy
                ul[5 * ii + 2, sl] = uz
                s_u = ux * ux + uy * uy + uz * uz
                ul[5 * ii + 3, sl] = _inv_norm(s_u)
                ul[5 * ii + 4, sl] = s_u
                return 0

            lax.fori_loop(0, NCG, ch_a, 0, unroll=False)
            return 0

        lax.fori_loop(0, UROWS, phase_a, 0, unroll=False)

        def phase_b(c, _):
            cs = c * LANES
            sl = pl.ds(cs, LANES)
            w0q = wl[0, sl]
            w1q = wl[1, sl]
            w2q = wl[2, sl]
            w0k = wl[3, sl]
            w1k = wl[4, sl]
            w2k = wl[5, sl]
            zero = jnp.zeros((LANES,), jnp.float32)

            def node_b(i, carry):
                aq, ak, bq, bk, cq, ck = carry
                n = n0 - 3 + i
                r = jnp.clip(n - sv, 0, VROWS - 1)
                vax = vecl[3 * r, sl]
                vay = vecl[3 * r + 1, sl]
                vaz = vecl[3 * r + 2, sl]
                uax = ul[5 * i, sl]
                uay = ul[5 * i + 1, sl]
                uaz = ul[5 * i + 2, sl]
                inv_ua = ul[5 * i + 3, sl]
                s_ua = ul[5 * i + 4, sl]
                xa = xl[jnp.clip(n - sx, 0, XROWS - 1), sl]
                q_fwd = zero
                k_fwd = zero
                rvq = []
                rvk = []
                for o in (1, 2, 3):
                    nb = n + o
                    ok = jnp.where((n >= 0) & (nb < N), 1.0, 0.0)
                    rb = jnp.clip(nb - sv, 0, VROWS - 1)
                    bx = vecl[3 * rb, sl] - vax
                    by = vecl[3 * rb + 1, sl] - vay
                    bz = vecl[3 * rb + 2, sl] - vaz
                    s_e = bx * bx + by * by + bz * bz
                    inv_e = _inv_norm(s_e)
                    ib = i + o
                    ubx = ul[5 * ib, sl]
                    uby = ul[5 * ib + 1, sl]
                    ubz = ul[5 * ib + 2, sl]
                    inv_ub = ul[5 * ib + 3, sl]
                    s_ub = ul[5 * ib + 4, sl]
                    p_a = uax * bx + uay * by + uaz * bz
                    p_b = ubx * bx + uby * by + ubz * bz
                    d_a = p_a * inv_e
                    d_b = p_b * inv_e
                    ang_ab = jnp.maximum(jnp.minimum(d_a * inv_ua, 1.0), -1.0)
                    ang_ba = jnp.maximum(
                        jnp.minimum(0.0 - d_b * inv_ub, 1.0), -1.0)
                    s_pa = s_ua - d_a * d_a
                    s_pb = s_ub - d_b * d_b
                    uaub = uax * ubx + uay * uby + uaz * ubz
                    dotp = uaub - d_a * d_b
                    spp = jnp.maximum(s_pa, EPS2) * jnp.maximum(s_pb, EPS2)
                    dih = dotp * _rsqrt_nr(spp)
                    dih = jnp.maximum(jnp.minimum(dih, 1.0), -1.0)
                    tq = dih * w1q + w2q
                    tk = dih * w1k + w2k
                    gq_ab = _sigmoid(tq + ang_ab * w0q)
                    gq_ba = _sigmoid(tq + ang_ba * w0q)
                    gk_ab = _sigmoid(tk + ang_ab * w0k)
                    gk_ba = _sigmoid(tk + ang_ba * w0k)
                    xb_ok = xl[jnp.clip(nb - sx, 0, XROWS - 1), sl] * ok
                    xa_ok = xa * ok
                    q_fwd = q_fwd + gq_ab * xb_ok
                    k_fwd = k_fwd + gk_ab * xb_ok
                    rvq.append(gq_ba * xa_ok)
                    rvk.append(gk_ba * xa_ok)
                ql[i, sl] = q_fwd + aq
                kl[i, sl] = k_fwd + ak
                return (bq + rvq[0], bk + rvk[0],
                        cq + rvq[1], ck + rvk[1],
                        rvq[2], rvk[2])

            lax.fori_loop(0, C + 3, node_b, (zero,) * 6, unroll=False)
            return 0

        lax.fori_loop(0, NCG, phase_b, 0, unroll=False)

        pltpu.sync_copy(ql.at[pl.ds(3, C)],
                        q_hbm.at[pl.ds(pl.multiple_of(bb + n0, 8), C)])
        pltpu.sync_copy(kl.at[pl.ds(3, C)],
                        k_hbm.at[pl.ds(pl.multiple_of(bb + n0, 8), C)])
        return 0

    lax.fori_loop(0, B, chunk_body, 0, unroll=False)


@jax.jit
def kernel(x_scalar, vec, w_angle_q, w_dih_q, b_q, w_angle_k, w_dih_k, b_k):
    vec_r = vec.reshape(B * N * 3, H)
    x_r = x_scalar.reshape(B * N, H)
    zrow = jnp.zeros_like(b_q)
    w_all = jnp.stack(
        [w_angle_q, w_dih_q, b_q, w_angle_k, w_dih_k, b_k, zrow, zrow])

    mesh = plsc.VectorSubcoreMesh(core_axis_name="c", subcore_axis_name="s")
    run = pl.kernel(
        _sc_body,
        out_type=(
            jax.ShapeDtypeStruct((B * N, H), jnp.float32),
            jax.ShapeDtypeStruct((B * N, H), jnp.float32),
        ),
        mesh=mesh,
        scratch_types=[
            pltpu.VMEM((3 * VROWS, H), jnp.float32),
            pltpu.VMEM((XROWS, H), jnp.float32),
            pltpu.VMEM((5 * UROWS, H), jnp.float32),
            pltpu.VMEM((C + 3, H), jnp.float32),
            pltpu.VMEM((C + 3, H), jnp.float32),
            pltpu.VMEM((8, H), jnp.float32),
        ],
    )
    q_r, k_r = run(vec_r, x_r, w_all)
    return q_r.reshape(B, N, H), k_r.reshape(B, N, H)


# speedup vs baseline: 2.5518x; 1.6654x over previous
import functools

import jax
import jax.numpy as jnp
from jax import lax
from jax.experimental import pallas as pl
from jax.experimental.pallas import tpu as pltpu
from jax.experimental.pallas import tpu_sc as plsc

B, N, H, W = 2, 2048, 128, 3
EPS = 1e-8
EPS2 = EPS * EPS
C = 64
NW = 32
LANES = 16
NCG = H // LANES
POFFS = (1, 2, 3)
OFFS = (-3, -2, -1, 1, 2, 3)
VROWS = C + 16
UROWS = C + 6
XROWS = C + 16


def _rsqrt_nr(s):
    i = lax.bitcast_convert_type(s, jnp.int32)
    y = lax.bitcast_convert_type(jnp.int32(0x5F3759DF) - (i >> 1), jnp.float32)
    for _ in range(2):
        y = y * (1.5 - 0.5 * s * y * y)
    return y


def _inv_norm(s):
    return _rsqrt_nr(jnp.maximum(s, EPS2))


def _sigmoid(z):
    return 1.0 / (1.0 + jnp.exp(-z))


def _sc_body(vec_hbm, x_hbm, w_hbm, q_hbm, k_hbm, vecl, xl, ul, ql, kl, wl):
    wid = lax.axis_index("s") * 2 + lax.axis_index("c")
    n0 = wid * C
    sv = jnp.clip(n0 - 8, 0, N - VROWS)
    sx = jnp.clip(n0 - 8, 0, N - XROWS)

    pltpu.sync_copy(w_hbm, wl)

    if True:
        bb = 0
        pltpu.sync_copy(
            vec_hbm.at[pl.ds(pl.multiple_of(3 * (bb + sv), 8), 3 * VROWS)],
            vecl)
        pltpu.sync_copy(
            x_hbm.at[pl.ds(pl.multiple_of(bb + sx, 8), XROWS)], xl)

        def phase_a(ii, _):
            n = n0 - 3 + ii
            r = jnp.clip(n - sv, 0, VROWS - 1)
            vi_ok = jnp.where((n >= 0) & (n < N), 1.0, 0.0)

            def ch_a(c, _):
                cs = c * LANES
                sl = pl.ds(cs, LANES)
                vix = vecl[3 * r, sl]
                viy = vecl[3 * r + 1, sl]
                viz = vecl[3 * r + 2, sl]
                ux = jnp.zeros((LANES,), jnp.float32)
                uy = jnp.zeros((LANES,), jnp.float32)
                uz = jnp.zeros((LANES,), jnp.float32)
                for o in OFFS:
                    n2 = n + o
                    r2 = jnp.clip(n2 - sv, 0, VROWS - 1)
                    bx = vecl[3 * r2, sl] - vix
                    by = vecl[3 * r2 + 1, sl] - viy
                    bz = vecl[3 * r2 + 2, sl] - viz
                    s = bx * bx + by * by + bz * bz
                    ok = vi_ok * jnp.where((n2 >= 0) & (n2 < N), 1.0, 0.0)
                    f = _inv_norm(s) * ok
                    ux = ux + bx * f
                    uy = uy + by * f
                    uz = uz + bz * f
                ul[5 * ii, sl] = ux
                ul[5 * ii + 1, sl] = uy
                ul[5 * ii + 2, sl] = uz
                s_u = ux * ux + uy * uy + uz * uz
                ul[5 * ii + 3, sl] = _inv_norm(s_u)
                ul[5 * ii + 4, sl] = s_u
                return 0

            lax.fori_loop(0, NCG, ch_a, 0, unroll=False)
            return 0

        lax.fori_loop(0, UROWS, phase_a, 0, unroll=False)

        def phase_b(c, _):
            cs = c * LANES
            sl = pl.ds(cs, LANES)
            w0q = wl[0, sl]
            w1q = wl[1, sl]
            w2q = wl[2, sl]
            w0k = wl[3, sl]
            w1k = wl[4, sl]
            w2k = wl[5, sl]
            zero = jnp.zeros((LANES,), jnp.float32)

            def node_b(i, carry):
                aq, ak, bq, bk, cq, ck = carry
                n = n0 - 3 + i
                r = jnp.clip(n - sv, 0, VROWS - 1)
                vax = vecl[3 * r, sl]
                vay = vecl[3 * r + 1, sl]
                vaz = vecl[3 * r + 2, sl]
                uax = ul[5 * i, sl]
                uay = ul[5 * i + 1, sl]
                uaz = ul[5 * i + 2, sl]
                inv_ua = ul[5 * i + 3, sl]
                s_ua = ul[5 * i + 4, sl]
                xa = xl[jnp.clip(n - sx, 0, XROWS - 1), sl]
                q_fwd = zero
                k_fwd = zero
                rvq = []
                rvk = []
                for o in (1, 2, 3):
                    nb = n + o
                    ok = jnp.where((n >= 0) & (nb < N), 1.0, 0.0)
                    rb = jnp.clip(nb - sv, 0, VROWS - 1)
                    bx = vecl[3 * rb, sl] - vax
                    by = vecl[3 * rb + 1, sl] - vay
                    bz = vecl[3 * rb + 2, sl] - vaz
                    s_e = bx * bx + by * by + bz * bz
                    inv_e = _inv_norm(s_e)
                    ib = i + o
                    ubx = ul[5 * ib, sl]
                    uby = ul[5 * ib + 1, sl]
                    ubz = ul[5 * ib + 2, sl]
                    inv_ub = ul[5 * ib + 3, sl]
                    s_ub = ul[5 * ib + 4, sl]
                    p_a = uax * bx + uay * by + uaz * bz
                    p_b = ubx * bx + uby * by + ubz * bz
                    d_a = p_a * inv_e
                    d_b = p_b * inv_e
                    ang_ab = jnp.maximum(jnp.minimum(d_a * inv_ua, 1.0), -1.0)
                    ang_ba = jnp.maximum(
                        jnp.minimum(0.0 - d_b * inv_ub, 1.0), -1.0)
                    s_pa = s_ua - d_a * d_a
                    s_pb = s_ub - d_b * d_b
                    uaub = uax * ubx + uay * uby + uaz * ubz
                    dotp = uaub - d_a * d_b
                    spp = jnp.maximum(s_pa, EPS2) * jnp.maximum(s_pb, EPS2)
                    dih = dotp * _rsqrt_nr(spp)
                    dih = jnp.maximum(jnp.minimum(dih, 1.0), -1.0)
                    tq = dih * w1q + w2q
                    tk = dih * w1k + w2k
                    gq_ab = _sigmoid(tq + ang_ab * w0q)
                    gq_ba = _sigmoid(tq + ang_ba * w0q)
                    gk_ab = _sigmoid(tk + ang_ab * w0k)
                    gk_ba = _sigmoid(tk + ang_ba * w0k)
                    xb_ok = xl[jnp.clip(nb - sx, 0, XROWS - 1), sl] * ok
                    xa_ok = xa * ok
                    q_fwd = q_fwd + gq_ab * xb_ok
                    k_fwd = k_fwd + gk_ab * xb_ok
                    rvq.append(gq_ba * xa_ok)
                    rvk.append(gk_ba * xa_ok)
                ql[i, sl] = q_fwd + aq
                kl[i, sl] = k_fwd + ak
                return (bq + rvq[0], bk + rvk[0],
                        cq + rvq[1], ck + rvk[1],
                        rvq[2], rvk[2])

            lax.fori_loop(0, C + 3, node_b, (zero,) * 6, unroll=False)
            return 0

        lax.fori_loop(0, NCG, phase_b, 0, unroll=False)

        pltpu.sync_copy(ql.at[pl.ds(3, C)],
                        q_hbm.at[pl.ds(pl.multiple_of(bb + n0, 8), C)])
        pltpu.sync_copy(kl.at[pl.ds(3, C)],
                        k_hbm.at[pl.ds(pl.multiple_of(bb + n0, 8), C)])


SC_ROWS = NW * C
TC_ROWS = B * N - SC_ROWS
UPAD = 8
PAD = 8


def _inv_norm_tc(s):
    return lax.rsqrt(jnp.maximum(s, EPS2))


def _tc_body(vx, vy, vz, xp, wr, q_ref, k_ref):
    ub = SC_ROWS - UPAD
    ru = TC_ROWS + UPAD + 3
    iu = lax.broadcasted_iota(jnp.int32, (ru, 1), 0)
    nu = (ub + iu) % N

    def vsl(ref, base, rows, o):
        return ref[pl.ds(base + o, rows), :]

    ux = jnp.zeros((ru, H), jnp.float32)
    uy = jnp.zeros((ru, H), jnp.float32)
    uz = jnp.zeros((ru, H), jnp.float32)
    vx0 = vsl(vx, ub, ru, 0)
    vy0 = vsl(vy, ub, ru, 0)
    vz0 = vsl(vz, ub, ru, 0)
    for o in OFFS:
        okm = ((nu + o >= 0) & (nu + o < N)).astype(jnp.float32)
        bx = vsl(vx, ub, ru, o) - vx0
        by = vsl(vy, ub, ru, o) - vy0
        bz = vsl(vz, ub, ru, o) - vz0
        s = bx * bx + by * by + bz * bz
        f = _inv_norm_tc(s) * okm
        ux = ux + bx * f
        uy = uy + by * f
        uz = uz + bz * f
    s_u = ux * ux + uy * uy + uz * uz
    inv_u = _inv_norm_tc(s_u)

    base = SC_ROWS
    lo = UPAD
    no = nu[lo:lo + TC_ROWS]
    vxc = vx0[lo:lo + TC_ROWS]
    vyc = vy0[lo:lo + TC_ROWS]
    vzc = vz0[lo:lo + TC_ROWS]
    uxc = ux[lo:lo + TC_ROWS]
    uyc = uy[lo:lo + TC_ROWS]
    uzc = uz[lo:lo + TC_ROWS]
    invuc = inv_u[lo:lo + TC_ROWS]
    s_uc = s_u[lo:lo + TC_ROWS]
    w0q = wr[0:1, :]
    w1q = wr[1:2, :]
    w2q = wr[2:3, :]
    w0k = wr[3:4, :]
    w1k = wr[4:5, :]
    w2k = wr[5:6, :]
    q = jnp.zeros((TC_ROWS, H), jnp.float32)
    k = jnp.zeros((TC_ROWS, H), jnp.float32)
    for o in OFFS:
        okm = ((no + o >= 0) & (no + o < N)).astype(jnp.float32)
        bx = vsl(vx, base, TC_ROWS, o) - vxc
        by = vsl(vy, base, TC_ROWS, o) - vyc
        bz = vsl(vz, base, TC_ROWS, o) - vzc
        s_e = bx * bx + by * by + bz * bz
        inv_e = _inv_norm_tc(s_e)
        ujx = ux[lo + o:lo + o + TC_ROWS]
        ujy = uy[lo + o:lo + o + TC_ROWS]
        ujz = uz[lo + o:lo + o + TC_ROWS]
        s_uj = s_u[lo + o:lo + o + TC_ROWS]
        p_i = uxc * bx + uyc * by + uzc * bz
        p_j = ujx * bx + ujy * by + ujz * bz
        d_i = p_i * inv_e
        d_j = p_j * inv_e
        ang = jnp.maximum(jnp.minimum(d_i * invuc, 1.0), -1.0)
        s_pi = s_uc - d_i * d_i
        s_pj = s_uj - d_j * d_j
        uiuj = uxc * ujx + uyc * ujy + uzc * ujz
        dotp = uiuj - d_i * d_j
        spp = jnp.maximum(s_pi, EPS2) * jnp.maximum(s_pj, EPS2)
        dih = dotp * lax.rsqrt(spp)
        dih = jnp.maximum(jnp.minimum(dih, 1.0), -1.0)
        zq = w2q + ang * w0q + dih * w1q
        zk = w2k + ang * w0k + dih * w1k
        gq = _sigmoid(zq)
        gk = _sigmoid(zk)
        xj = vsl(xp, base, TC_ROWS, o) * okm
        q = q + gq * xj
        k = k + gk * xj
    q_ref[...] = q
    k_ref[...] = k


@jax.jit
def kernel(x_scalar, vec, w_angle_q, w_dih_q, b_q, w_angle_k, w_dih_k, b_k):
    vec_r = vec.reshape(B * N * 3, H)
    x_r = x_scalar.reshape(B * N, H)
    zrow = jnp.zeros_like(b_q)
    w_all = jnp.stack(
        [w_angle_q, w_dih_q, b_q, w_angle_k, w_dih_k, b_k, zrow, zrow])

    mesh = plsc.VectorSubcoreMesh(core_axis_name="c", subcore_axis_name="s")
    run = pl.kernel(
        _sc_body,
        out_type=(
            jax.ShapeDtypeStruct((SC_ROWS, H), jnp.float32),
            jax.ShapeDtypeStruct((SC_ROWS, H), jnp.float32),
        ),
        mesh=mesh,
        scratch_types=[
            pltpu.VMEM((3 * VROWS, H), jnp.float32),
            pltpu.VMEM((XROWS, H), jnp.float32),
            pltpu.VMEM((5 * UROWS, H), jnp.float32),
            pltpu.VMEM((C + 3, H), jnp.float32),
            pltpu.VMEM((C + 3, H), jnp.float32),
            pltpu.VMEM((8, H), jnp.float32),
        ],
    )
    q_sc, k_sc = run(vec_r, x_r, w_all)

    pad = jnp.zeros((PAD, H), jnp.float32)
    vxp = jnp.concatenate([vec[:, :, 0, :].reshape(B * N, H), pad])
    vyp = jnp.concatenate([vec[:, :, 1, :].reshape(B * N, H), pad])
    vzp = jnp.concatenate([vec[:, :, 2, :].reshape(B * N, H), pad])
    xp = jnp.concatenate([x_r, pad])
    q_tc, k_tc = pl.pallas_call(
        _tc_body,
        out_shape=(
            jax.ShapeDtypeStruct((TC_ROWS, H), jnp.float32),
            jax.ShapeDtypeStruct((TC_ROWS, H), jnp.float32),
        ),
    )(vxp, vyp, vzp, xp, w_all)

    q_r = jnp.concatenate([q_sc, q_tc])
    k_r = jnp.concatenate([k_sc, k_tc])
    return q_r.reshape(B, N, H), k_r.reshape(B, N, H)


# speedup vs baseline: 4.2864x; 1.6798x over previous
import functools

import jax
import jax.numpy as jnp
from jax import lax
from jax.experimental import pallas as pl
from jax.experimental.pallas import tpu as pltpu
from jax.experimental.pallas import tpu_sc as plsc

B, N, H, W = 2, 2048, 128, 3
EPS = 1e-8
EPS2 = EPS * EPS
C = 16
NW = 32
LANES = 16
NCG = H // LANES
POFFS = (1, 2, 3)
OFFS = (-3, -2, -1, 1, 2, 3)
VROWS = C + 16
UROWS = C + 6
XROWS = C + 16


def _rsqrt_nr(s):
    i = lax.bitcast_convert_type(s, jnp.int32)
    y = lax.bitcast_convert_type(jnp.int32(0x5F3759DF) - (i >> 1), jnp.float32)
    for _ in range(2):
        y = y * (1.5 - 0.5 * s * y * y)
    return y


def _inv_norm(s):
    return _rsqrt_nr(jnp.maximum(s, EPS2))


def _sigmoid(z):
    return 1.0 / (1.0 + jnp.exp(-z))


def _sc_body(vec_hbm, x_hbm, w_hbm, q_hbm, k_hbm, vecl, xl, ul, ql, kl, wl):
    wid = lax.axis_index("s") * 2 + lax.axis_index("c")
    n0 = wid * C
    sv = jnp.clip(n0 - 8, 0, N - VROWS)
    sx = jnp.clip(n0 - 8, 0, N - XROWS)

    pltpu.sync_copy(w_hbm, wl)

    if True:
        bb = 0
        pltpu.sync_copy(
            vec_hbm.at[pl.ds(pl.multiple_of(3 * (bb + sv), 8), 3 * VROWS)],
            vecl)
        pltpu.sync_copy(
            x_hbm.at[pl.ds(pl.multiple_of(bb + sx, 8), XROWS)], xl)

        def phase_a(ii, _):
            n = n0 - 3 + ii
            r = jnp.clip(n - sv, 0, VROWS - 1)
            vi_ok = jnp.where((n >= 0) & (n < N), 1.0, 0.0)

            def ch_a(c, _):
                cs = c * LANES
                sl = pl.ds(cs, LANES)
                vix = vecl[3 * r, sl]
                viy = vecl[3 * r + 1, sl]
                viz = vecl[3 * r + 2, sl]
                ux = jnp.zeros((LANES,), jnp.float32)
                uy = jnp.zeros((LANES,), jnp.float32)
                uz = jnp.zeros((LANES,), jnp.float32)
                for o in OFFS:
                    n2 = n + o
                    r2 = jnp.clip(n2 - sv, 0, VROWS - 1)
                    bx = vecl[3 * r2, sl] - vix
                    by = vecl[3 * r2 + 1, sl] - viy
                    bz = vecl[3 * r2 + 2, sl] - viz
                    s = bx * bx + by * by + bz * bz
                    ok = vi_ok * jnp.where((n2 >= 0) & (n2 < N), 1.0, 0.0)
                    f = _inv_norm(s) * ok
                    ux = ux + bx * f
                    uy = uy + by * f
                    uz = uz + bz * f
                ul[5 * ii, sl] = ux
                ul[5 * ii + 1, sl] = uy
                ul[5 * ii + 2, sl] = uz
                s_u = ux * ux + uy * uy + uz * uz
                ul[5 * ii + 3, sl] = _inv_norm(s_u)
                ul[5 * ii + 4, sl] = s_u
                return 0

            lax.fori_loop(0, NCG, ch_a, 0, unroll=False)
            return 0

        lax.fori_loop(0, UROWS, phase_a, 0, unroll=False)

        def phase_b(c, _):
            cs = c * LANES
            sl = pl.ds(cs, LANES)
            w0q = wl[0, sl]
            w1q = wl[1, sl]
            w2q = wl[2, sl]
            w0k = wl[3, sl]
            w1k = wl[4, sl]
            w2k = wl[5, sl]
            zero = jnp.zeros((LANES,), jnp.float32)

            def node_b(i, carry):
                aq, ak, bq, bk, cq, ck = carry
                n = n0 - 3 + i
                r = jnp.clip(n - sv, 0, VROWS - 1)
                vax = vecl[3 * r, sl]
                vay = vecl[3 * r + 1, sl]
                vaz = vecl[3 * r + 2, sl]
                uax = ul[5 * i, sl]
                uay = ul[5 * i + 1, sl]
                uaz = ul[5 * i + 2, sl]
                inv_ua = ul[5 * i + 3, sl]
                s_ua = ul[5 * i + 4, sl]
                xa = xl[jnp.clip(n - sx, 0, XROWS - 1), sl]
                q_fwd = zero
                k_fwd = zero
                rvq = []
                rvk = []
                for o in (1, 2, 3):
                    nb = n + o
                    ok = jnp.where((n >= 0) & (nb < N), 1.0, 0.0)
                    rb = jnp.clip(nb - sv, 0, VROWS - 1)
                    bx = vecl[3 * rb, sl] - vax
                    by = vecl[3 * rb + 1, sl] - vay
                    bz = vecl[3 * rb + 2, sl] - vaz
                    s_e = bx * bx + by * by + bz * bz
                    inv_e = _inv_norm(s_e)
                    ib = i + o
                    ubx = ul[5 * ib, sl]
                    uby = ul[5 * ib + 1, sl]
                    ubz = ul[5 * ib + 2, sl]
                    inv_ub = ul[5 * ib + 3, sl]
                    s_ub = ul[5 * ib + 4, sl]
                    p_a = uax * bx + uay * by + uaz * bz
                    p_b = ubx * bx + uby * by + ubz * bz
                    d_a = p_a * inv_e
                    d_b = p_b * inv_e
                    ang_ab = jnp.maximum(jnp.minimum(d_a * inv_ua, 1.0), -1.0)
                    ang_ba = jnp.maximum(
                        jnp.minimum(0.0 - d_b * inv_ub, 1.0), -1.0)
                    s_pa = s_ua - d_a * d_a
                    s_pb = s_ub - d_b * d_b
                    uaub = uax * ubx + uay * uby + uaz * ubz
                    dotp = uaub - d_a * d_b
                    spp = jnp.maximum(s_pa, EPS2) * jnp.maximum(s_pb, EPS2)
                    dih = dotp * _rsqrt_nr(spp)
                    dih = jnp.maximum(jnp.minimum(dih, 1.0), -1.0)
                    tq = dih * w1q + w2q
                    tk = dih * w1k + w2k
                    gq_ab = _sigmoid(tq + ang_ab * w0q)
                    gq_ba = _sigmoid(tq + ang_ba * w0q)
                    gk_ab = _sigmoid(tk + ang_ab * w0k)
                    gk_ba = _sigmoid(tk + ang_ba * w0k)
                    xb_ok = xl[jnp.clip(nb - sx, 0, XROWS - 1), sl] * ok
                    xa_ok = xa * ok
                    q_fwd = q_fwd + gq_ab * xb_ok
                    k_fwd = k_fwd + gk_ab * xb_ok
                    rvq.append(gq_ba * xa_ok)
                    rvk.append(gk_ba * xa_ok)
                ql[i, sl] = q_fwd + aq
                kl[i, sl] = k_fwd + ak
                return (bq + rvq[0], bk + rvk[0],
                        cq + rvq[1], ck + rvk[1],
                        rvq[2], rvk[2])

            lax.fori_loop(0, C + 3, node_b, (zero,) * 6, unroll=False)
            return 0

        lax.fori_loop(0, NCG, phase_b, 0, unroll=False)

        pltpu.sync_copy(ql.at[pl.ds(3, C)],
                        q_hbm.at[pl.ds(pl.multiple_of(bb + n0, 8), C)])
        pltpu.sync_copy(kl.at[pl.ds(3, C)],
                        k_hbm.at[pl.ds(pl.multiple_of(bb + n0, 8), C)])


SC_ROWS = NW * C
TC_ROWS = B * N - SC_ROWS
UPAD = 8
PAD = 8


def _inv_norm_tc(s):
    return lax.rsqrt(jnp.maximum(s, EPS2))


def _tc_body(vx, vy, vz, xp, wr, q_ref, k_ref):
    ub = SC_ROWS - UPAD
    ru = TC_ROWS + UPAD + 3
    iu = lax.broadcasted_iota(jnp.int32, (ru, 1), 0)
    nu = (ub + iu) % N

    def vsl(ref, base, rows, o):
        return ref[pl.ds(base + o, rows), :]

    ux = jnp.zeros((ru, H), jnp.float32)
    uy = jnp.zeros((ru, H), jnp.float32)
    uz = jnp.zeros((ru, H), jnp.float32)
    vx0 = vsl(vx, ub, ru, 0)
    vy0 = vsl(vy, ub, ru, 0)
    vz0 = vsl(vz, ub, ru, 0)
    for o in OFFS:
        okm = ((nu + o >= 0) & (nu + o < N)).astype(jnp.float32)
        bx = vsl(vx, ub, ru, o) - vx0
        by = vsl(vy, ub, ru, o) - vy0
        bz = vsl(vz, ub, ru, o) - vz0
        s = bx * bx + by * by + bz * bz
        f = _inv_norm_tc(s) * okm
        ux = ux + bx * f
        uy = uy + by * f
        uz = uz + bz * f
    s_u = ux * ux + uy * uy + uz * uz
    inv_u = _inv_norm_tc(s_u)

    base = SC_ROWS
    lo = UPAD
    no = nu[lo:lo + TC_ROWS]
    vxc = vx0[lo:lo + TC_ROWS]
    vyc = vy0[lo:lo + TC_ROWS]
    vzc = vz0[lo:lo + TC_ROWS]
    uxc = ux[lo:lo + TC_ROWS]
    uyc = uy[lo:lo + TC_ROWS]
    uzc = uz[lo:lo + TC_ROWS]
    invuc = inv_u[lo:lo + TC_ROWS]
    s_uc = s_u[lo:lo + TC_ROWS]
    w0q = wr[0:1, :]
    w1q = wr[1:2, :]
    w2q = wr[2:3, :]
    w0k = wr[3:4, :]
    w1k = wr[4:5, :]
    w2k = wr[5:6, :]
    q = jnp.zeros((TC_ROWS, H), jnp.float32)
    k = jnp.zeros((TC_ROWS, H), jnp.float32)
    for o in OFFS:
        okm = ((no + o >= 0) & (no + o < N)).astype(jnp.float32)
        bx = vsl(vx, base, TC_ROWS, o) - vxc
        by = vsl(vy, base, TC_ROWS, o) - vyc
        bz = vsl(vz, base, TC_ROWS, o) - vzc
        s_e = bx * bx + by * by + bz * bz
        inv_e = _inv_norm_tc(s_e)
        ujx = ux[lo + o:lo + o + TC_ROWS]
        ujy = uy[lo + o:lo + o + TC_ROWS]
        ujz = uz[lo + o:lo + o + TC_ROWS]
        s_uj = s_u[lo + o:lo + o + TC_ROWS]
        p_i = uxc * bx + uyc * by + uzc * bz
        p_j = ujx * bx + ujy * by + ujz * bz
        d_i = p_i * inv_e
        d_j = p_j * inv_e
        ang = jnp.maximum(jnp.minimum(d_i * invuc, 1.0), -1.0)
        s_pi = s_uc - d_i * d_i
        s_pj = s_uj - d_j * d_j
        uiuj = uxc * ujx + uyc * ujy + uzc * ujz
        dotp = uiuj - d_i * d_j
        spp = jnp.maximum(s_pi, EPS2) * jnp.maximum(s_pj, EPS2)
        dih = dotp * lax.rsqrt(spp)
        dih = jnp.maximum(jnp.minimum(dih, 1.0), -1.0)
        zq = w2q + ang * w0q + dih * w1q
        zk = w2k + ang * w0k + dih * w1k
        gq = _sigmoid(zq)
        gk = _sigmoid(zk)
        xj = vsl(xp, base, TC_ROWS, o) * okm
        q = q + gq * xj
        k = k + gk * xj
    q_ref[...] = q
    k_ref[...] = k


@jax.jit
def kernel(x_scalar, vec, w_angle_q, w_dih_q, b_q, w_angle_k, w_dih_k, b_k):
    vec_r = vec.reshape(B * N * 3, H)
    x_r = x_scalar.reshape(B * N, H)
    zrow = jnp.zeros_like(b_q)
    w_all = jnp.stack(
        [w_angle_q, w_dih_q, b_q, w_angle_k, w_dih_k, b_k, zrow, zrow])

    mesh = plsc.VectorSubcoreMesh(core_axis_name="c", subcore_axis_name="s")
    run = pl.kernel(
        _sc_body,
        out_type=(
            jax.ShapeDtypeStruct((SC_ROWS, H), jnp.float32),
            jax.ShapeDtypeStruct((SC_ROWS, H), jnp.float32),
        ),
        mesh=mesh,
        scratch_types=[
            pltpu.VMEM((3 * VROWS, H), jnp.float32),
            pltpu.VMEM((XROWS, H), jnp.float32),
            pltpu.VMEM((5 * UROWS, H), jnp.float32),
            pltpu.VMEM((C + 3, H), jnp.float32),
            pltpu.VMEM((C + 3, H), jnp.float32),
            pltpu.VMEM((8, H), jnp.float32),
        ],
    )
    q_sc, k_sc = run(vec_r, x_r, w_all)

    pad = jnp.zeros((PAD, H), jnp.float32)
    vxp = jnp.concatenate([vec[:, :, 0, :].reshape(B * N, H), pad])
    vyp = jnp.concatenate([vec[:, :, 1, :].reshape(B * N, H), pad])
    vzp = jnp.concatenate([vec[:, :, 2, :].reshape(B * N, H), pad])
    xp = jnp.concatenate([x_r, pad])
    q_tc, k_tc = _tc_call(vxp, vyp, vzp, xp, w_all)

    q_r = jnp.concatenate([q_sc, q_tc])
    k_r = jnp.concatenate([k_sc, k_tc])
    return q_r.reshape(B, N, H), k_r.reshape(B, N, H)


def _tc_call(vxp, vyp, vzp, xp, w_all):
    return pl.pallas_call(
        _tc_body,
        out_shape=(
            jax.ShapeDtypeStruct((TC_ROWS, H), jnp.float32),
            jax.ShapeDtypeStruct((TC_ROWS, H), jnp.float32),
        ),
    )(vxp, vyp, vzp, xp, w_all)


# speedup vs baseline: 5.2328x; 1.2208x over previous
import functools

import jax
import jax.numpy as jnp
from jax import lax
from jax.experimental import pallas as pl
from jax.experimental.pallas import tpu as pltpu
from jax.experimental.pallas import tpu_sc as plsc

B, N, H, W = 2, 2048, 128, 3
EPS = 1e-8
EPS2 = EPS * EPS
C = 8
NW = 32
LANES = 16
NCG = H // LANES
POFFS = (1, 2, 3)
OFFS = (-3, -2, -1, 1, 2, 3)
VROWS = C + 16
UROWS = C + 6
XROWS = C + 16


def _rsqrt_nr(s):
    i = lax.bitcast_convert_type(s, jnp.int32)
    y = lax.bitcast_convert_type(jnp.int32(0x5F3759DF) - (i >> 1), jnp.float32)
    for _ in range(2):
        y = y * (1.5 - 0.5 * s * y * y)
    return y


def _inv_norm(s):
    return _rsqrt_nr(jnp.maximum(s, EPS2))


def _sigmoid(z):
    return 1.0 / (1.0 + jnp.exp(-z))


def _sc_body(vec_hbm, x_hbm, w_hbm, q_hbm, k_hbm, vecl, xl, ul, ql, kl, wl):
    wid = lax.axis_index("s") * 2 + lax.axis_index("c")
    n0 = wid * C
    sv = jnp.clip(n0 - 8, 0, N - VROWS)
    sx = jnp.clip(n0 - 8, 0, N - XROWS)

    pltpu.sync_copy(w_hbm, wl)

    if True:
        bb = 0
        pltpu.sync_copy(
            vec_hbm.at[pl.ds(pl.multiple_of(3 * (bb + sv), 8), 3 * VROWS)],
            vecl)
        pltpu.sync_copy(
            x_hbm.at[pl.ds(pl.multiple_of(bb + sx, 8), XROWS)], xl)

        def phase_a(ii, _):
            n = n0 - 3 + ii
            r = jnp.clip(n - sv, 0, VROWS - 1)
            vi_ok = jnp.where((n >= 0) & (n < N), 1.0, 0.0)

            def ch_a(c, _):
                cs = c * LANES
                sl = pl.ds(cs, LANES)
                vix = vecl[3 * r, sl]
                viy = vecl[3 * r + 1, sl]
                viz = vecl[3 * r + 2, sl]
                ux = jnp.zeros((LANES,), jnp.float32)
                uy = jnp.zeros((LANES,), jnp.float32)
                uz = jnp.zeros((LANES,), jnp.float32)
                for o in OFFS:
                    n2 = n + o
                    r2 = jnp.clip(n2 - sv, 0, VROWS - 1)
                    bx = vecl[3 * r2, sl] - vix
                    by = vecl[3 * r2 + 1, sl] - viy
                    bz = vecl[3 * r2 + 2, sl] - viz
                    s = bx * bx + by * by + bz * bz
                    ok = vi_ok * jnp.where((n2 >= 0) & (n2 < N), 1.0, 0.0)
                    f = _inv_norm(s) * ok
                    ux = ux + bx * f
                    uy = uy + by * f
                    uz = uz + bz * f
                ul[5 * ii, sl] = ux
                ul[5 * ii + 1, sl] = uy
                ul[5 * ii + 2, sl] = uz
                s_u = ux * ux + uy * uy + uz * uz
                ul[5 * ii + 3, sl] = _inv_norm(s_u)
                ul[5 * ii + 4, sl] = s_u
                return 0

            lax.fori_loop(0, NCG, ch_a, 0, unroll=False)
            return 0

        lax.fori_loop(0, UROWS, phase_a, 0, unroll=False)

        def phase_b(c, _):
            cs = c * LANES
            sl = pl.ds(cs, LANES)
            w0q = wl[0, sl]
            w1q = wl[1, sl]
            w2q = wl[2, sl]
            w0k = wl[3, sl]
            w1k = wl[4, sl]
            w2k = wl[5, sl]
            zero = jnp.zeros((LANES,), jnp.float32)

            def node_b(i, carry):
                aq, ak, bq, bk, cq, ck = carry
                n = n0 - 3 + i
                r = jnp.clip(n - sv, 0, VROWS - 1)
                vax = vecl[3 * r, sl]
                vay = vecl[3 * r + 1, sl]
                vaz = vecl[3 * r + 2, sl]
                uax = ul[5 * i, sl]
                uay = ul[5 * i + 1, sl]
                uaz = ul[5 * i + 2, sl]
                inv_ua = ul[5 * i + 3, sl]
                s_ua = ul[5 * i + 4, sl]
                xa = xl[jnp.clip(n - sx, 0, XROWS - 1), sl]
                q_fwd = zero
                k_fwd = zero
                rvq = []
                rvk = []
                for o in (1, 2, 3):
                    nb = n + o
                    ok = jnp.where((n >= 0) & (nb < N), 1.0, 0.0)
                    rb = jnp.clip(nb - sv, 0, VROWS - 1)
                    bx = vecl[3 * rb, sl] - vax
                    by = vecl[3 * rb + 1, sl] - vay
                    bz = vecl[3 * rb + 2, sl] - vaz
                    s_e = bx * bx + by * by + bz * bz
                    inv_e = _inv_norm(s_e)
                    ib = i + o
                    ubx = ul[5 * ib, sl]
                    uby = ul[5 * ib + 1, sl]
                    ubz = ul[5 * ib + 2, sl]
                    inv_ub = ul[5 * ib + 3, sl]
                    s_ub = ul[5 * ib + 4, sl]
                    p_a = uax * bx + uay * by + uaz * bz
                    p_b = ubx * bx + uby * by + ubz * bz
                    d_a = p_a * inv_e
                    d_b = p_b * inv_e
                    ang_ab = jnp.maximum(jnp.minimum(d_a * inv_ua, 1.0), -1.0)
                    ang_ba = jnp.maximum(
                        jnp.minimum(0.0 - d_b * inv_ub, 1.0), -1.0)
                    s_pa = s_ua - d_a * d_a
                    s_pb = s_ub - d_b * d_b
                    uaub = uax * ubx + uay * uby + uaz * ubz
                    dotp = uaub - d_a * d_b
                    spp = jnp.maximum(s_pa, EPS2) * jnp.maximum(s_pb, EPS2)
                    dih = dotp * _rsqrt_nr(spp)
                    dih = jnp.maximum(jnp.minimum(dih, 1.0), -1.0)
                    tq = dih * w1q + w2q
                    tk = dih * w1k + w2k
                    gq_ab = _sigmoid(tq + ang_ab * w0q)
                    gq_ba = _sigmoid(tq + ang_ba * w0q)
                    gk_ab = _sigmoid(tk + ang_ab * w0k)
                    gk_ba = _sigmoid(tk + ang_ba * w0k)
                    xb_ok = xl[jnp.clip(nb - sx, 0, XROWS - 1), sl] * ok
                    xa_ok = xa * ok
                    q_fwd = q_fwd + gq_ab * xb_ok
                    k_fwd = k_fwd + gk_ab * xb_ok
                    rvq.append(gq_ba * xa_ok)
                    rvk.append(gk_ba * xa_ok)
                ql[i, sl] = q_fwd + aq
                kl[i, sl] = k_fwd + ak
                return (bq + rvq[0], bk + rvk[0],
                        cq + rvq[1], ck + rvk[1],
                        rvq[2], rvk[2])

            lax.fori_loop(0, C + 3, node_b, (zero,) * 6, unroll=False)
            return 0

        lax.fori_loop(0, NCG, phase_b, 0, unroll=False)

        pltpu.sync_copy(ql.at[pl.ds(3, C)],
                        q_hbm.at[pl.ds(pl.multiple_of(bb + n0, 8), C)])
        pltpu.sync_copy(kl.at[pl.ds(3, C)],
                        k_hbm.at[pl.ds(pl.multiple_of(bb + n0, 8), C)])


SC_ROWS = NW * C
TC_ROWS = B * N - SC_ROWS
UPAD = 8
PAD = 8


def _inv_norm_tc(s):
    return lax.rsqrt(jnp.maximum(s, EPS2))


def _tc_body(vx, vy, vz, xp, wr, q_ref, k_ref):
    ub = SC_ROWS - UPAD
    ru = TC_ROWS + UPAD + 3
    iu = lax.broadcasted_iota(jnp.int32, (ru, 1), 0)
    nu = (ub + iu) % N

    def vsl(ref, base, rows, o):
        return ref[pl.ds(base + o, rows), :]

    ux = jnp.zeros((ru, H), jnp.float32)
    uy = jnp.zeros((ru, H), jnp.float32)
    uz = jnp.zeros((ru, H), jnp.float32)
    vx0 = vsl(vx, ub, ru, 0)
    vy0 = vsl(vy, ub, ru, 0)
    vz0 = vsl(vz, ub, ru, 0)
    for o in OFFS:
        okm = ((nu + o >= 0) & (nu + o < N)).astype(jnp.float32)
        bx = vsl(vx, ub, ru, o) - vx0
        by = vsl(vy, ub, ru, o) - vy0
        bz = vsl(vz, ub, ru, o) - vz0
        s = bx * bx + by * by + bz * bz
        f = _inv_norm_tc(s) * okm
        ux = ux + bx * f
        uy = uy + by * f
        uz = uz + bz * f
    s_u = ux * ux + uy * uy + uz * uz
    inv_u = _inv_norm_tc(s_u)

    base = SC_ROWS
    lo = UPAD
    no = nu[lo:lo + TC_ROWS]
    vxc = vx0[lo:lo + TC_ROWS]
    vyc = vy0[lo:lo + TC_ROWS]
    vzc = vz0[lo:lo + TC_ROWS]
    uxc = ux[lo:lo + TC_ROWS]
    uyc = uy[lo:lo + TC_ROWS]
    uzc = uz[lo:lo + TC_ROWS]
    invuc = inv_u[lo:lo + TC_ROWS]
    s_uc = s_u[lo:lo + TC_ROWS]
    w0q = wr[0:1, :]
    w1q = wr[1:2, :]
    w2q = wr[2:3, :]
    w0k = wr[3:4, :]
    w1k = wr[4:5, :]
    w2k = wr[5:6, :]
    q = jnp.zeros((TC_ROWS, H), jnp.float32)
    k = jnp.zeros((TC_ROWS, H), jnp.float32)
    for o in OFFS:
        okm = ((no + o >= 0) & (no + o < N)).astype(jnp.float32)
        bx = vsl(vx, base, TC_ROWS, o) - vxc
        by = vsl(vy, base, TC_ROWS, o) - vyc
        bz = vsl(vz, base, TC_ROWS, o) - vzc
        s_e = bx * bx + by * by + bz * bz
        inv_e = _inv_norm_tc(s_e)
        ujx = ux[lo + o:lo + o + TC_ROWS]
        ujy = uy[lo + o:lo + o + TC_ROWS]
        ujz = uz[lo + o:lo + o + TC_ROWS]
        s_uj = s_u[lo + o:lo + o + TC_ROWS]
        p_i = uxc * bx + uyc * by + uzc * bz
        p_j = ujx * bx + ujy * by + ujz * bz
        d_i = p_i * inv_e
        d_j = p_j * inv_e
        ang = jnp.maximum(jnp.minimum(d_i * invuc, 1.0), -1.0)
        s_pi = s_uc - d_i * d_i
        s_pj = s_uj - d_j * d_j
        uiuj = uxc * ujx + uyc * ujy + uzc * ujz
        dotp = uiuj - d_i * d_j
        spp = jnp.maximum(s_pi, EPS2) * jnp.maximum(s_pj, EPS2)
        dih = dotp * lax.rsqrt(spp)
        dih = jnp.maximum(jnp.minimum(dih, 1.0), -1.0)
        zq = w2q + ang * w0q + dih * w1q
        zk = w2k + ang * w0k + dih * w1k
        gq = _sigmoid(zq)
        gk = _sigmoid(zk)
        xj = vsl(xp, base, TC_ROWS, o) * okm
        q = q + gq * xj
        k = k + gk * xj
    q_ref[...] = q
    k_ref[...] = k


@jax.jit
def kernel(x_scalar, vec, w_angle_q, w_dih_q, b_q, w_angle_k, w_dih_k, b_k):
    x_full = x_scalar.reshape(B * N, H)
    vec_r = vec.reshape(B * N * 3, H)[:3 * (SC_ROWS + 8)]
    x_r = x_full[:SC_ROWS + 8]
    zrow = jnp.zeros_like(b_q)
    w_all = jnp.stack(
        [w_angle_q, w_dih_q, b_q, w_angle_k, w_dih_k, b_k, zrow, zrow])

    mesh = plsc.VectorSubcoreMesh(core_axis_name="c", subcore_axis_name="s")
    run = pl.kernel(
        _sc_body,
        out_type=(
            jax.ShapeDtypeStruct((SC_ROWS, H), jnp.float32),
            jax.ShapeDtypeStruct((SC_ROWS, H), jnp.float32),
        ),
        mesh=mesh,
        scratch_types=[
            pltpu.VMEM((3 * VROWS, H), jnp.float32),
            pltpu.VMEM((XROWS, H), jnp.float32),
            pltpu.VMEM((5 * UROWS, H), jnp.float32),
            pltpu.VMEM((C + 3, H), jnp.float32),
            pltpu.VMEM((C + 3, H), jnp.float32),
            pltpu.VMEM((8, H), jnp.float32),
        ],
    )
    q_sc, k_sc = run(vec_r, x_r, w_all)

    pad = jnp.zeros((PAD, H), jnp.float32)
    vxp = jnp.concatenate([vec[:, :, 0, :].reshape(B * N, H), pad])
    vyp = jnp.concatenate([vec[:, :, 1, :].reshape(B * N, H), pad])
    vzp = jnp.concatenate([vec[:, :, 2, :].reshape(B * N, H), pad])
    xp = jnp.concatenate([x_full, pad])
    q_tc, k_tc = _tc_call(vxp, vyp, vzp, xp, w_all)

    q_r = jnp.concatenate([q_sc, q_tc])
    k_r = jnp.concatenate([k_sc, k_tc])
    return q_r.reshape(B, N, H), k_r.reshape(B, N, H)


def _tc_call(vxp, vyp, vzp, xp, w_all):
    return pl.pallas_call(
        _tc_body,
        out_shape=(
            jax.ShapeDtypeStruct((TC_ROWS, H), jnp.float32),
            jax.ShapeDtypeStruct((TC_ROWS, H), jnp.float32),
        ),
    )(vxp, vyp, vzp, xp, w_all)


# speedup vs baseline: 5.2879x; 1.0105x over previous
import functools

import jax
import jax.numpy as jnp
from jax import lax
from jax.experimental import pallas as pl
from jax.experimental.pallas import tpu as pltpu
from jax.experimental.pallas import tpu_sc as plsc

B, N, H, W = 2, 2048, 128, 3
EPS = 1e-8
EPS2 = EPS * EPS
C = 8
NW = 32
LANES = 16
NCG = H // LANES
POFFS = (1, 2, 3)
OFFS = (-3, -2, -1, 1, 2, 3)
VROWS = C + 16
UROWS = C + 6
XROWS = C + 16


def _rsqrt_nr(s):
    i = lax.bitcast_convert_type(s, jnp.int32)
    y = lax.bitcast_convert_type(jnp.int32(0x5F3759DF) - (i >> 1), jnp.float32)
    for _ in range(2):
        y = y * (1.5 - 0.5 * s * y * y)
    return y


def _inv_norm(s):
    return _rsqrt_nr(jnp.maximum(s, EPS2))


def _sigmoid(z):
    return 1.0 / (1.0 + jnp.exp(-z))


def _sc_body(vec_hbm, x_hbm, w_hbm, q_hbm, k_hbm, vecl, xl, ul, ql, kl, wl):
    wid = lax.axis_index("s") * 2 + lax.axis_index("c")
    n0 = wid * C
    sv = jnp.clip(n0 - 8, 0, N - VROWS)
    sx = jnp.clip(n0 - 8, 0, N - XROWS)

    pltpu.sync_copy(w_hbm, wl)

    if True:
        bb = 0
        pltpu.sync_copy(
            vec_hbm.at[pl.ds(pl.multiple_of(3 * (bb + sv), 8), 3 * VROWS)],
            vecl)
        pltpu.sync_copy(
            x_hbm.at[pl.ds(pl.multiple_of(bb + sx, 8), XROWS)], xl)

        def phase_a(ii, _):
            n = n0 - 3 + ii
            r = jnp.clip(n - sv, 0, VROWS - 1)
            vi_ok = jnp.where((n >= 0) & (n < N), 1.0, 0.0)

            def ch_a(c, _):
                cs = c * LANES
                sl = pl.ds(cs, LANES)
                vix = vecl[3 * r, sl]
                viy = vecl[3 * r + 1, sl]
                viz = vecl[3 * r + 2, sl]
                ux = jnp.zeros((LANES,), jnp.float32)
                uy = jnp.zeros((LANES,), jnp.float32)
                uz = jnp.zeros((LANES,), jnp.float32)
                for o in OFFS:
                    n2 = n + o
                    r2 = jnp.clip(n2 - sv, 0, VROWS - 1)
                    bx = vecl[3 * r2, sl] - vix
                    by = vecl[3 * r2 + 1, sl] - viy
                    bz = vecl[3 * r2 + 2, sl] - viz
                    s = bx * bx + by * by + bz * bz
                    ok = vi_ok * jnp.where((n2 >= 0) & (n2 < N), 1.0, 0.0)
                    f = _inv_norm(s) * ok
                    ux = ux + bx * f
                    uy = uy + by * f
                    uz = uz + bz * f
                ul[5 * ii, sl] = ux
                ul[5 * ii + 1, sl] = uy
                ul[5 * ii + 2, sl] = uz
                s_u = ux * ux + uy * uy + uz * uz
                ul[5 * ii + 3, sl] = _inv_norm(s_u)
                ul[5 * ii + 4, sl] = s_u
                return 0

            lax.fori_loop(0, NCG, ch_a, 0, unroll=False)
            return 0

        lax.fori_loop(0, UROWS, phase_a, 0, unroll=False)

        def phase_b(c, _):
            cs = c * LANES
            sl = pl.ds(cs, LANES)
            w0q = wl[0, sl]
            w1q = wl[1, sl]
            w2q = wl[2, sl]
            w0k = wl[3, sl]
            w1k = wl[4, sl]
            w2k = wl[5, sl]
            zero = jnp.zeros((LANES,), jnp.float32)

            def node_b(i, carry):
                aq, ak, bq, bk, cq, ck = carry
                n = n0 - 3 + i
                r = jnp.clip(n - sv, 0, VROWS - 1)
                vax = vecl[3 * r, sl]
                vay = vecl[3 * r + 1, sl]
                vaz = vecl[3 * r + 2, sl]
                uax = ul[5 * i, sl]
                uay = ul[5 * i + 1, sl]
                uaz = ul[5 * i + 2, sl]
                inv_ua = ul[5 * i + 3, sl]
                s_ua = ul[5 * i + 4, sl]
                xa = xl[jnp.clip(n - sx, 0, XROWS - 1), sl]
                q_fwd = zero
                k_fwd = zero
                rvq = []
                rvk = []
                for o in (1, 2, 3):
                    nb = n + o
                    ok = jnp.where((n >= 0) & (nb < N), 1.0, 0.0)
                    rb = jnp.clip(nb - sv, 0, VROWS - 1)
                    bx = vecl[3 * rb, sl] - vax
                    by = vecl[3 * rb + 1, sl] - vay
                    bz = vecl[3 * rb + 2, sl] - vaz
                    s_e = bx * bx + by * by + bz * bz
                    inv_e = _inv_norm(s_e)
                    ib = i + o
                    ubx = ul[5 * ib, sl]
                    uby = ul[5 * ib + 1, sl]
                    ubz = ul[5 * ib + 2, sl]
                    inv_ub = ul[5 * ib + 3, sl]
                    s_ub = ul[5 * ib + 4, sl]
                    p_a = uax * bx + uay * by + uaz * bz
                    p_b = ubx * bx + uby * by + ubz * bz
                    d_a = p_a * inv_e
                    d_b = p_b * inv_e
                    ang_ab = jnp.maximum(jnp.minimum(d_a * inv_ua, 1.0), -1.0)
                    ang_ba = jnp.maximum(
                        jnp.minimum(0.0 - d_b * inv_ub, 1.0), -1.0)
                    s_pa = s_ua - d_a * d_a
                    s_pb = s_ub - d_b * d_b
                    uaub = uax * ubx + uay * uby + uaz * ubz
                    dotp = uaub - d_a * d_b
                    spp = jnp.maximum(s_pa, EPS2) * jnp.maximum(s_pb, EPS2)
                    dih = dotp * _rsqrt_nr(spp)
                    dih = jnp.maximum(jnp.minimum(dih, 1.0), -1.0)
                    tq = dih * w1q + w2q
                    tk = dih * w1k + w2k
                    gq_ab = _sigmoid(tq + ang_ab * w0q)
                    gq_ba = _sigmoid(tq + ang_ba * w0q)
                    gk_ab = _sigmoid(tk + ang_ab * w0k)
                    gk_ba = _sigmoid(tk + ang_ba * w0k)
                    xb_ok = xl[jnp.clip(nb - sx, 0, XROWS - 1), sl] * ok
                    xa_ok = xa * ok
                    q_fwd = q_fwd + gq_ab * xb_ok
                    k_fwd = k_fwd + gk_ab * xb_ok
                    rvq.append(gq_ba * xa_ok)
                    rvk.append(gk_ba * xa_ok)
                ql[i, sl] = q_fwd + aq
                kl[i, sl] = k_fwd + ak
                return (bq + rvq[0], bk + rvk[0],
                        cq + rvq[1], ck + rvk[1],
                        rvq[2], rvk[2])

            lax.fori_loop(0, C + 3, node_b, (zero,) * 6, unroll=False)
            return 0

        lax.fori_loop(0, NCG, phase_b, 0, unroll=False)

        pltpu.sync_copy(ql.at[pl.ds(3, C)],
                        q_hbm.at[pl.ds(pl.multiple_of(bb + n0, 8), C)])
        pltpu.sync_copy(kl.at[pl.ds(3, C)],
                        k_hbm.at[pl.ds(pl.multiple_of(bb + n0, 8), C)])


SC_ROWS = NW * C
TC_ROWS = B * N - SC_ROWS
UPAD = 8
PAD = 8


def _inv_norm_tc(s):
    return lax.rsqrt(jnp.maximum(s, EPS2))


def _tc_body(vx, vy, vz, xp, wr, q_ref, k_ref):
    ub = SC_ROWS - UPAD
    ru = TC_ROWS + UPAD + 3
    P = ru - 3
    lo = UPAD
    iu = lax.broadcasted_iota(jnp.int32, (ru, 1), 0)
    nu = (ub + iu) % N

    def vsl(ref, base, rows, o):
        return ref[pl.ds(base + o, rows), :]

    eb = ub - 3
    re = ru + 3
    ie = lax.broadcasted_iota(jnp.int32, (re, 1), 0)
    ne = (eb + ie) % N
    ex = {}
    ey = {}
    ez = {}
    vx0e = vsl(vx, eb, re, 0)
    vy0e = vsl(vy, eb, re, 0)
    vz0e = vsl(vz, eb, re, 0)
    for o in (1, 2, 3):
        okm = ((ne + o < N)).astype(jnp.float32)
        bx = vsl(vx, eb, re, o) - vx0e
        by = vsl(vy, eb, re, o) - vy0e
        bz = vsl(vz, eb, re, o) - vz0e
        s = bx * bx + by * by + bz * bz
        f = _inv_norm_tc(s) * okm
        ex[o] = bx * f
        ey[o] = by * f
        ez[o] = bz * f
    ux = jnp.zeros((ru, H), jnp.float32)
    uy = jnp.zeros((ru, H), jnp.float32)
    uz = jnp.zeros((ru, H), jnp.float32)
    for o in (1, 2, 3):
        ux = ux + ex[o][3:3 + ru] - ex[o][3 - o:3 - o + ru]
        uy = uy + ey[o][3:3 + ru] - ey[o][3 - o:3 - o + ru]
        uz = uz + ez[o][3:3 + ru] - ez[o][3 - o:3 - o + ru]
    s_u = ux * ux + uy * uy + uz * uz
    inv_u = _inv_norm_tc(s_u)

    w0q = wr[0:1, :]
    w1q = wr[1:2, :]
    w2q = wr[2:3, :]
    w0k = wr[3:4, :]
    w1k = wr[4:5, :]
    w2k = wr[5:6, :]
    q = jnp.zeros((TC_ROWS, H), jnp.float32)
    k = jnp.zeros((TC_ROWS, H), jnp.float32)
    nup = nu[:P]
    for o in (1, 2, 3):
        okm = ((nup + o < N)).astype(jnp.float32)
        bx = vsl(vx, ub, P, o) - vsl(vx, ub, P, 0)
        by = vsl(vy, ub, P, o) - vsl(vy, ub, P, 0)
        bz = vsl(vz, ub, P, o) - vsl(vz, ub, P, 0)
        s_e = bx * bx + by * by + bz * bz
        inv_e = _inv_norm_tc(s_e)
        uax = ux[:P]
        uay = uy[:P]
        uaz = uz[:P]
        ubx = ux[o:o + P]
        uby = uy[o:o + P]
        ubz = uz[o:o + P]
        p_a = uax * bx + uay * by + uaz * bz
        p_b = ubx * bx + uby * by + ubz * bz
        d_a = p_a * inv_e
        d_b = p_b * inv_e
        ang_ab = jnp.maximum(jnp.minimum(d_a * inv_u[:P], 1.0), -1.0)
        ang_ba = jnp.maximum(
            jnp.minimum(0.0 - d_b * inv_u[o:o + P], 1.0), -1.0)
        s_pa = s_u[:P] - d_a * d_a
        s_pb = s_u[o:o + P] - d_b * d_b
        uaub = uax * ubx + uay * uby + uaz * ubz
        dotp = uaub - d_a * d_b
        spp = jnp.maximum(s_pa, EPS2) * jnp.maximum(s_pb, EPS2)
        dih = dotp * lax.rsqrt(spp)
        dih = jnp.maximum(jnp.minimum(dih, 1.0), -1.0)
        tq = dih * w1q + w2q
        tk = dih * w1k + w2k
        xa_ok = vsl(xp, ub, P, 0) * okm
        xb_ok = vsl(xp, ub, P, o) * okm
        fq = _sigmoid(tq + ang_ab * w0q) * xb_ok
        fk = _sigmoid(tk + ang_ab * w0k) * xb_ok
        rq = _sigmoid(tq + ang_ba * w0q) * xa_ok
        rk = _sigmoid(tk + ang_ba * w0k) * xa_ok
        q = q + fq[lo:lo + TC_ROWS] + rq[lo - o:lo - o + TC_ROWS]
        k = k + fk[lo:lo + TC_ROWS] + rk[lo - o:lo - o + TC_ROWS]
    q_ref[...] = q
    k_ref[...] = k


@jax.jit
def kernel(x_scalar, vec, w_angle_q, w_dih_q, b_q, w_angle_k, w_dih_k, b_k):
    x_full = x_scalar.reshape(B * N, H)
    vec_r = vec.reshape(B * N * 3, H)[:3 * (SC_ROWS + 8)]
    x_r = x_full[:SC_ROWS + 8]
    zrow = jnp.zeros_like(b_q)
    w_all = jnp.stack(
        [w_angle_q, w_dih_q, b_q, w_angle_k, w_dih_k, b_k, zrow, zrow])

    mesh = plsc.VectorSubcoreMesh(core_axis_name="c", subcore_axis_name="s")
    run = pl.kernel(
        _sc_body,
        out_type=(
            jax.ShapeDtypeStruct((SC_ROWS, H), jnp.float32),
            jax.ShapeDtypeStruct((SC_ROWS, H), jnp.float32),
        ),
        mesh=mesh,
        scratch_types=[
            pltpu.VMEM((3 * VROWS, H), jnp.float32),
            pltpu.VMEM((XROWS, H), jnp.float32),
            pltpu.VMEM((5 * UROWS, H), jnp.float32),
            pltpu.VMEM((C + 3, H), jnp.float32),
            pltpu.VMEM((C + 3, H), jnp.float32),
            pltpu.VMEM((8, H), jnp.float32),
        ],
    )
    q_sc, k_sc = run(vec_r, x_r, w_all)

    pad = jnp.zeros((PAD, H), jnp.float32)
    vxp = jnp.concatenate([vec[:, :, 0, :].reshape(B * N, H), pad])
    vyp = jnp.concatenate([vec[:, :, 1, :].reshape(B * N, H), pad])
    vzp = jnp.concatenate([vec[:, :, 2, :].reshape(B * N, H), pad])
    xp = jnp.concatenate([x_full, pad])
    q_tc, k_tc = _tc_call(vxp, vyp, vzp, xp, w_all)

    q_r = jnp.concatenate([q_sc, q_tc])
    k_r = jnp.concatenate([k_sc, k_tc])
    return q_r.reshape(B, N, H), k_r.reshape(B, N, H)


def _tc_call(vxp, vyp, vzp, xp, w_all):
    return pl.pallas_call(
        _tc_body,
        out_shape=(
            jax.ShapeDtypeStruct((TC_ROWS, H), jnp.float32),
            jax.ShapeDtypeStruct((TC_ROWS, H), jnp.float32),
        ),
    )(vxp, vyp, vzp, xp, w_all)


# speedup vs baseline: 6.3797x; 1.2065x over previous
import functools

import jax
import jax.numpy as jnp
from jax import lax
from jax.experimental import pallas as pl
from jax.experimental.pallas import tpu as pltpu
from jax.experimental.pallas import tpu_sc as plsc

B, N, H, W = 2, 2048, 128, 3
EPS = 1e-8
EPS2 = EPS * EPS
C = 16
NW = 32
LANES = 16
NCG = H // LANES
POFFS = (1, 2, 3)
OFFS = (-3, -2, -1, 1, 2, 3)
VROWS = C + 16
UROWS = C + 6
XROWS = C + 16


def _rsqrt_nr(s):
    i = lax.bitcast_convert_type(s, jnp.int32)
    y = lax.bitcast_convert_type(jnp.int32(0x5F3759DF) - (i >> 1), jnp.float32)
    for _ in range(2):
        y = y * (1.5 - 0.5 * s * y * y)
    return y


def _inv_norm(s):
    return _rsqrt_nr(jnp.maximum(s, EPS2))


def _sigmoid(z):
    return 1.0 / (1.0 + jnp.exp(-z))


def _sc_body(vec_hbm, x_hbm, w_hbm, q_hbm, k_hbm, vecl, xl, ul, ql, kl, wl):
    wid = lax.axis_index("s") * 2 + lax.axis_index("c")
    n0 = wid * C
    sv = jnp.clip(n0 - 8, 0, N - VROWS)
    sx = jnp.clip(n0 - 8, 0, N - XROWS)

    pltpu.sync_copy(w_hbm, wl)

    if True:
        bb = 0
        pltpu.sync_copy(
            vec_hbm.at[pl.ds(pl.multiple_of(3 * (bb + sv), 8), 3 * VROWS)],
            vecl)
        pltpu.sync_copy(
            x_hbm.at[pl.ds(pl.multiple_of(bb + sx, 8), XROWS)], xl)

        def phase_a(ii, _):
            n = n0 - 3 + ii
            r = jnp.clip(n - sv, 0, VROWS - 1)
            vi_ok = jnp.where((n >= 0) & (n < N), 1.0, 0.0)

            def ch_a(c, _):
                cs = c * LANES
                sl = pl.ds(cs, LANES)
                vix = vecl[3 * r, sl]
                viy = vecl[3 * r + 1, sl]
                viz = vecl[3 * r + 2, sl]
                ux = jnp.zeros((LANES,), jnp.float32)
                uy = jnp.zeros((LANES,), jnp.float32)
                uz = jnp.zeros((LANES,), jnp.float32)
                for o in OFFS:
                    n2 = n + o
                    r2 = jnp.clip(n2 - sv, 0, VROWS - 1)
                    bx = vecl[3 * r2, sl] - vix
                    by = vecl[3 * r2 + 1, sl] - viy
                    bz = vecl[3 * r2 + 2, sl] - viz
                    s = bx * bx + by * by + bz * bz
                    ok = vi_ok * jnp.where((n2 >= 0) & (n2 < N), 1.0, 0.0)
                    f = _inv_norm(s) * ok
                    ux = ux + bx * f
                    uy = uy + by * f
                    uz = uz + bz * f
                ul[5 * ii, sl] = ux
                ul[5 * ii + 1, sl] = uy
                ul[5 * ii + 2, sl] = uz
                s_u = ux * ux + uy * uy + uz * uz
                ul[5 * ii + 3, sl] = _inv_norm(s_u)
                ul[5 * ii + 4, sl] = s_u
                return 0

            lax.fori_loop(0, NCG, ch_a, 0, unroll=False)
            return 0

        lax.fori_loop(0, UROWS, phase_a, 0, unroll=False)

        def phase_b(c, _):
            cs = c * LANES
            sl = pl.ds(cs, LANES)
            w0q = wl[0, sl]
            w1q = wl[1, sl]
            w2q = wl[2, sl]
            w0k = wl[3, sl]
            w1k = wl[4, sl]
            w2k = wl[5, sl]
            zero = jnp.zeros((LANES,), jnp.float32)

            def node_b(i, carry):
                aq, ak, bq, bk, cq, ck = carry
                n = n0 - 3 + i
                r = jnp.clip(n - sv, 0, VROWS - 1)
                vax = vecl[3 * r, sl]
                vay = vecl[3 * r + 1, sl]
                vaz = vecl[3 * r + 2, sl]
                uax = ul[5 * i, sl]
                uay = ul[5 * i + 1, sl]
                uaz = ul[5 * i + 2, sl]
                inv_ua = ul[5 * i + 3, sl]
                s_ua = ul[5 * i + 4, sl]
                xa = xl[jnp.clip(n - sx, 0, XROWS - 1), sl]
                q_fwd = zero
                k_fwd = zero
                rvq = []
                rvk = []
                for o in (1, 2, 3):
                    nb = n + o
                    ok = jnp.where((n >= 0) & (nb < N), 1.0, 0.0)
                    rb = jnp.clip(nb - sv, 0, VROWS - 1)
                    bx = vecl[3 * rb, sl] - vax
                    by = vecl[3 * rb + 1, sl] - vay
                    bz = vecl[3 * rb + 2, sl] - vaz
                    s_e = bx * bx + by * by + bz * bz
                    inv_e = _inv_norm(s_e)
                    ib = i + o
                    ubx = ul[5 * ib, sl]
                    uby = ul[5 * ib + 1, sl]
                    ubz = ul[5 * ib + 2, sl]
                    inv_ub = ul[5 * ib + 3, sl]
                    s_ub = ul[5 * ib + 4, sl]
                    p_a = uax * bx + uay * by + uaz * bz
                    p_b = ubx * bx + uby * by + ubz * bz
                    d_a = p_a * inv_e
                    d_b = p_b * inv_e
                    ang_ab = jnp.maximum(jnp.minimum(d_a * inv_ua, 1.0), -1.0)
                    ang_ba = jnp.maximum(
                        jnp.minimum(0.0 - d_b * inv_ub, 1.0), -1.0)
                    s_pa = s_ua - d_a * d_a
                    s_pb = s_ub - d_b * d_b
                    uaub = uax * ubx + uay * uby + uaz * ubz
                    dotp = uaub - d_a * d_b
                    spp = jnp.maximum(s_pa, EPS2) * jnp.maximum(s_pb, EPS2)
                    dih = dotp * _rsqrt_nr(spp)
                    dih = jnp.maximum(jnp.minimum(dih, 1.0), -1.0)
                    tq = dih * w1q + w2q
                    tk = dih * w1k + w2k
                    gq_ab = _sigmoid(tq + ang_ab * w0q)
                    gq_ba = _sigmoid(tq + ang_ba * w0q)
                    gk_ab = _sigmoid(tk + ang_ab * w0k)
                    gk_ba = _sigmoid(tk + ang_ba * w0k)
                    xb_ok = xl[jnp.clip(nb - sx, 0, XROWS - 1), sl] * ok
                    xa_ok = xa * ok
                    q_fwd = q_fwd + gq_ab * xb_ok
                    k_fwd = k_fwd + gk_ab * xb_ok
                    rvq.append(gq_ba * xa_ok)
                    rvk.append(gk_ba * xa_ok)
                ql[i, sl] = q_fwd + aq
                kl[i, sl] = k_fwd + ak
                return (bq + rvq[0], bk + rvk[0],
                        cq + rvq[1], ck + rvk[1],
                        rvq[2], rvk[2])

            lax.fori_loop(0, C + 3, node_b, (zero,) * 6, unroll=False)
            return 0

        lax.fori_loop(0, NCG, phase_b, 0, unroll=False)

        pltpu.sync_copy(ql.at[pl.ds(3, C)],
                        q_hbm.at[pl.ds(pl.multiple_of(bb + n0, 8), C)])
        pltpu.sync_copy(kl.at[pl.ds(3, C)],
                        k_hbm.at[pl.ds(pl.multiple_of(bb + n0, 8), C)])


SC_ROWS = NW * C
TC_ROWS = B * N - SC_ROWS
UPAD = 8
PAD = 8


def _inv_norm_tc(s):
    return lax.rsqrt(jnp.maximum(s, EPS2))


def _tc_body(vx, vy, vz, xp, wr, q_ref, k_ref):
    ub = SC_ROWS - UPAD
    ru = TC_ROWS + UPAD + 3
    P = ru - 3
    lo = UPAD
    iu = lax.broadcasted_iota(jnp.int32, (ru, 1), 0)
    nu = (ub + iu) % N

    def vsl(ref, base, rows, o):
        return ref[pl.ds(base + o, rows), :]

    eb = ub - 3
    re = ru + 3
    ie = lax.broadcasted_iota(jnp.int32, (re, 1), 0)
    ne = (eb + ie) % N
    ex = {}
    ey = {}
    ez = {}
    vx0e = vsl(vx, eb, re, 0)
    vy0e = vsl(vy, eb, re, 0)
    vz0e = vsl(vz, eb, re, 0)
    for o in (1, 2, 3):
        okm = ((ne + o < N)).astype(jnp.float32)
        bx = vsl(vx, eb, re, o) - vx0e
        by = vsl(vy, eb, re, o) - vy0e
        bz = vsl(vz, eb, re, o) - vz0e
        s = bx * bx + by * by + bz * bz
        f = _inv_norm_tc(s) * okm
        ex[o] = bx * f
        ey[o] = by * f
        ez[o] = bz * f
    ux = jnp.zeros((ru, H), jnp.float32)
    uy = jnp.zeros((ru, H), jnp.float32)
    uz = jnp.zeros((ru, H), jnp.float32)
    for o in (1, 2, 3):
        ux = ux + ex[o][3:3 + ru] - ex[o][3 - o:3 - o + ru]
        uy = uy + ey[o][3:3 + ru] - ey[o][3 - o:3 - o + ru]
        uz = uz + ez[o][3:3 + ru] - ez[o][3 - o:3 - o + ru]
    s_u = ux * ux + uy * uy + uz * uz
    inv_u = _inv_norm_tc(s_u)

    w0q = wr[0:1, :]
    w1q = wr[1:2, :]
    w2q = wr[2:3, :]
    w0k = wr[3:4, :]
    w1k = wr[4:5, :]
    w2k = wr[5:6, :]
    q = jnp.zeros((TC_ROWS, H), jnp.float32)
    k = jnp.zeros((TC_ROWS, H), jnp.float32)
    nup = nu[:P]
    for o in (1, 2, 3):
        okm = ((nup + o < N)).astype(jnp.float32)
        eax = ex[o][3:3 + P]
        eay = ey[o][3:3 + P]
        eaz = ez[o][3:3 + P]
        uax = ux[:P]
        uay = uy[:P]
        uaz = uz[:P]
        ubx = ux[o:o + P]
        uby = uy[o:o + P]
        ubz = uz[o:o + P]
        d_a = uax * eax + uay * eay + uaz * eaz
        d_b = ubx * eax + uby * eay + ubz * eaz
        ang_ab = jnp.maximum(jnp.minimum(d_a * inv_u[:P], 1.0), -1.0)
        ang_ba = jnp.maximum(
            jnp.minimum(0.0 - d_b * inv_u[o:o + P], 1.0), -1.0)
        s_pa = s_u[:P] - d_a * d_a
        s_pb = s_u[o:o + P] - d_b * d_b
        uaub = uax * ubx + uay * uby + uaz * ubz
        dotp = uaub - d_a * d_b
        spp = jnp.maximum(s_pa, EPS2) * jnp.maximum(s_pb, EPS2)
        dih = dotp * lax.rsqrt(spp)
        dih = jnp.maximum(jnp.minimum(dih, 1.0), -1.0)
        tq = dih * w1q + w2q
        tk = dih * w1k + w2k
        xa_ok = vsl(xp, ub, P, 0) * okm
        xb_ok = vsl(xp, ub, P, o) * okm
        fq = _sigmoid(tq + ang_ab * w0q) * xb_ok
        fk = _sigmoid(tk + ang_ab * w0k) * xb_ok
        rq = _sigmoid(tq + ang_ba * w0q) * xa_ok
        rk = _sigmoid(tk + ang_ba * w0k) * xa_ok
        q = q + fq[lo:lo + TC_ROWS] + rq[lo - o:lo - o + TC_ROWS]
        k = k + fk[lo:lo + TC_ROWS] + rk[lo - o:lo - o + TC_ROWS]
    q_ref[...] = q
    k_ref[...] = k


@jax.jit
def kernel(x_scalar, vec, w_angle_q, w_dih_q, b_q, w_angle_k, w_dih_k, b_k):
    x_full = x_scalar.reshape(B * N, H)
    vec_r = vec.reshape(B * N * 3, H)[:3 * (SC_ROWS + 8)]
    x_r = x_full[:SC_ROWS + 8]
    zrow = jnp.zeros_like(b_q)
    w_all = jnp.stack(
        [w_angle_q, w_dih_q, b_q, w_angle_k, w_dih_k, b_k, zrow, zrow])

    mesh = plsc.VectorSubcoreMesh(core_axis_name="c", subcore_axis_name="s")
    run = pl.kernel(
        _sc_body,
        out_type=(
            jax.ShapeDtypeStruct((SC_ROWS, H), jnp.float32),
            jax.ShapeDtypeStruct((SC_ROWS, H), jnp.float32),
        ),
        mesh=mesh,
        scratch_types=[
            pltpu.VMEM((3 * VROWS, H), jnp.float32),
            pltpu.VMEM((XROWS, H), jnp.float32),
            pltpu.VMEM((5 * UROWS, H), jnp.float32),
            pltpu.VMEM((C + 3, H), jnp.float32),
            pltpu.VMEM((C + 3, H), jnp.float32),
            pltpu.VMEM((8, H), jnp.float32),
        ],
    )
    q_sc, k_sc = run(vec_r, x_r, w_all)

    pad = jnp.zeros((PAD, H), jnp.float32)
    vxp = jnp.concatenate([vec[:, :, 0, :].reshape(B * N, H), pad])
    vyp = jnp.concatenate([vec[:, :, 1, :].reshape(B * N, H), pad])
    vzp = jnp.concatenate([vec[:, :, 2, :].reshape(B * N, H), pad])
    xp = jnp.concatenate([x_full, pad])
    q_tc, k_tc = _tc_call(vxp, vyp, vzp, xp, w_all)

    q_r = jnp.concatenate([q_sc, q_tc])
    k_r = jnp.concatenate([k_sc, k_tc])
    return q_r.reshape(B, N, H), k_r.reshape(B, N, H)


def _tc_call(vxp, vyp, vzp, xp, w_all):
    return pl.pallas_call(
        _tc_body,
        out_shape=(
            jax.ShapeDtypeStruct((TC_ROWS, H), jnp.float32),
            jax.ShapeDtypeStruct((TC_ROWS, H), jnp.float32),
        ),
    )(vxp, vyp, vzp, xp, w_all)
